# baseline (speedup 1.0000x reference)
import functools

import jax
import jax.numpy as jnp
from jax import lax
from jax.experimental import pallas as pl
from jax.experimental.pallas import tpu as pltpu
from jax.experimental.pallas import tpu_sc as plsc

_TOP_N = 2000
_TOP_N_POST = 300
_NMS_THRESH = 0.7
_ROWS = 16
_COLS = 128
_PAD = _ROWS * _COLS
_TD = 128


def _decode_nms_body(dx_ref, dy_ref, dw_ref, dh_ref,
                     ax1_ref, ay1_ref, ax2_ref, ay2_ref,
                     x1_o, y1_o, x2_o, y2_o, keep_o):
    ax1 = ax1_ref[...]
    ay1 = ay1_ref[...]
    ax2 = ax2_ref[...]
    ay2 = ay2_ref[...]
    aw = ax2 - ax1 + 1.0
    ah = ay2 - ay1 + 1.0
    acx = ax1 + 0.5 * aw
    acy = ay1 + 0.5 * ah
    cx = dx_ref[...] * aw + acx
    cy = dy_ref[...] * ah + acy
    w = jnp.exp(jnp.clip(dw_ref[...], -10.0, 10.0)) * aw
    h = jnp.exp(jnp.clip(dh_ref[...], -10.0, 10.0)) * ah
    x1 = cx - 0.5 * w
    y1 = cy - 0.5 * h
    x2 = cx + 0.5 * w
    y2 = cy + 0.5 * h
    x1_o[...] = x1
    y1_o[...] = y1
    x2_o[...] = x2
    y2_o[...] = y2
    area = jnp.maximum(x2 - x1, 0.0) * jnp.maximum(y2 - y1, 0.0)

    lane = jax.lax.broadcasted_iota(jnp.int32, (1, _COLS), 1)
    flat_idx = (jax.lax.broadcasted_iota(jnp.int32, (_ROWS, _COLS), 0) * _COLS
                + jax.lax.broadcasted_iota(jnp.int32, (_ROWS, _COLS), 1))

    keep = jnp.ones((_ROWS, _COLS), jnp.float32)
    for r in range(_ROWS):
        x1r = x1[r:r + 1, :]
        y1r = y1[r:r + 1, :]
        x2r = x2[r:r + 1, :]
        y2r = y2[r:r + 1, :]
        base = r * _COLS

        def body(c, keep, x1r=x1r, y1r=y1r, x2r=x2r, y2r=y2r, r=r, base=base):
            sel = lane == c

            def pick(row):
                return jnp.sum(jnp.where(sel, row, 0.0))

            bx1 = pick(x1r)
            by1 = pick(y1r)
            bx2 = pick(x2r)
            by2 = pick(y2r)
            bkeep = pick(keep[r:r + 1, :])
            barea = (jnp.maximum(bx2 - bx1, 0.0)
                     * jnp.maximum(by2 - by1, 0.0))

            ix1 = jnp.maximum(bx1, x1)
            iy1 = jnp.maximum(by1, y1)
            ix2 = jnp.minimum(bx2, x2)
            iy2 = jnp.minimum(by2, y2)
            iw = jnp.maximum(ix2 - ix1, 0.0)
            ih = jnp.maximum(iy2 - iy1, 0.0)
            inter = iw * ih
            union = barea + area - inter
            iou = inter / (union + 1e-8)
            sup = (iou > _NMS_THRESH) & (flat_idx > base + c) & (bkeep > 0.5)
            return jnp.where(sup, 0.0, keep)

        keep = jax.lax.fori_loop(0, _COLS, body, keep, unroll=False)
    keep_o[...] = keep


def _sc_gather_rows(table, idx):
    info = plsc.get_sparse_core_info()
    nw = info.num_cores * info.num_subcores
    b_per_w = _PAD // nw
    mesh = plsc.VectorSubcoreMesh(core_axis_name="c", subcore_axis_name="s")

    @functools.partial(
        pl.kernel, mesh=mesh,
        out_type=jax.ShapeDtypeStruct((_PAD, _TD), jnp.float32),
        scratch_types=[
            pltpu.VMEM((b_per_w,), jnp.int32),
            pltpu.VMEM((b_per_w, _TD), jnp.float32),
            pltpu.SemaphoreType.DMA,
        ],
    )
    def gather_k(table_hbm, idx_hbm, out_hbm, idx_v, rows_v, sem):
        wid = lax.axis_index("s") * info.num_cores + lax.axis_index("c")
        base = wid * b_per_w
        pltpu.sync_copy(idx_hbm.at[pl.ds(base, b_per_w)], idx_v)
        pltpu.async_copy(table_hbm.at[idx_v], rows_v, sem).wait()
        pltpu.sync_copy(rows_v, out_hbm.at[pl.ds(base, b_per_w)])

    return gather_k(table, idx)


@jax.jit
def kernel(rpn_box, rpn_prob, anchors):
    objness = 1.0 - rpn_prob[:, 0]
    scores, inds = jax.lax.top_k(objness, _TOP_N)

    n = rpn_box.shape[0]
    table = jnp.concatenate(
        [rpn_box, anchors, jnp.zeros((n, _TD - 8), jnp.float32)], axis=1)
    idx = jnp.pad(inds, (0, _PAD - _TOP_N)).astype(jnp.int32)
    rows = _sc_gather_rows(table, idx)

    ins = [rows[:, j].reshape(_ROWS, _COLS) for j in range(8)]

    shp = jax.ShapeDtypeStruct((_ROWS, _COLS), jnp.float32)
    x1, y1, x2, y2, keepf = pl.pallas_call(
        _decode_nms_body,
        out_shape=[shp] * 5,
    )(*ins)

    decoded = jnp.stack(
        [x1.reshape(-1), y1.reshape(-1), x2.reshape(-1), y2.reshape(-1)],
        axis=1)[:_TOP_N]
    keep = keepf.reshape(-1)[:_TOP_N] > 0.5
    masked = jnp.where(keep, scores, -1e9)
    final_scores, fi = jax.lax.top_k(masked, _TOP_N_POST)
    final_boxes = jnp.take(decoded, fi, axis=0)
    return jnp.concatenate([final_boxes, final_scores[:, None]], axis=1)


# speedup vs baseline: 1.0969x; 1.0969x over previous
import functools

import jax
import jax.numpy as jnp
from jax import lax
from jax.experimental import pallas as pl
from jax.experimental.pallas import tpu as pltpu
from jax.experimental.pallas import tpu_sc as plsc

_TOP_N = 2000
_TOP_N_POST = 300
_NMS_THRESH = 0.7
_ROWS = 16
_COLS = 128
_PAD = _ROWS * _COLS
_TD = 128


def _decode_nms_body(dx_ref, dy_ref, dw_ref, dh_ref,
                     ax1_ref, ay1_ref, ax2_ref, ay2_ref,
                     x1_o, y1_o, x2_o, y2_o, keep_o):
    ax1 = ax1_ref[...]
    ay1 = ay1_ref[...]
    ax2 = ax2_ref[...]
    ay2 = ay2_ref[...]
    aw = ax2 - ax1 + 1.0
    ah = ay2 - ay1 + 1.0
    acx = ax1 + 0.5 * aw
    acy = ay1 + 0.5 * ah
    cx = dx_ref[...] * aw + acx
    cy = dy_ref[...] * ah + acy
    w = jnp.exp(jnp.clip(dw_ref[...], -10.0, 10.0)) * aw
    h = jnp.exp(jnp.clip(dh_ref[...], -10.0, 10.0)) * ah
    x1 = cx - 0.5 * w
    y1 = cy - 0.5 * h
    x2 = cx + 0.5 * w
    y2 = cy + 0.5 * h
    x1_o[...] = x1
    y1_o[...] = y1
    x2_o[...] = x2
    y2_o[...] = y2
    area = jnp.maximum(x2 - x1, 0.0) * jnp.maximum(y2 - y1, 0.0)

    lane = jax.lax.broadcasted_iota(jnp.int32, (1, _COLS), 1)
    flat_idx = (jax.lax.broadcasted_iota(jnp.int32, (_ROWS, _COLS), 0) * _COLS
                + jax.lax.broadcasted_iota(jnp.int32, (_ROWS, _COLS), 1))

    keep = jnp.ones((_ROWS, _COLS), jnp.float32)
    for r in range(_ROWS):
        x1r = x1[r:r + 1, :]
        y1r = y1[r:r + 1, :]
        x2r = x2[r:r + 1, :]
        y2r = y2[r:r + 1, :]
        base = r * _COLS

        def body(c, keep, x1r=x1r, y1r=y1r, x2r=x2r, y2r=y2r, r=r, base=base):
            def pick(row):
                return pltpu.roll(row, -c, axis=1)[:, 0:1]

            bx1 = pick(x1r)
            by1 = pick(y1r)
            bx2 = pick(x2r)
            by2 = pick(y2r)
            bkeep = pick(keep[r:r + 1, :])
            barea = (jnp.maximum(bx2 - bx1, 0.0)
                     * jnp.maximum(by2 - by1, 0.0))

            ix1 = jnp.maximum(bx1, x1)
            iy1 = jnp.maximum(by1, y1)
            ix2 = jnp.minimum(bx2, x2)
            iy2 = jnp.minimum(by2, y2)
            iw = jnp.maximum(ix2 - ix1, 0.0)
            ih = jnp.maximum(iy2 - iy1, 0.0)
            inter = iw * ih
            union = barea + area - inter
            iou = inter / (union + 1e-8)
            sup = (iou > _NMS_THRESH) & (flat_idx > base + c) & (bkeep > 0.5)
            return jnp.where(sup, 0.0, keep)

        keep = jax.lax.fori_loop(0, _COLS, body, keep, unroll=False)
    keep_o[...] = keep


def _sc_gather_rows(table, idx):
    info = plsc.get_sparse_core_info()
    nw = info.num_cores * info.num_subcores
    b_per_w = _PAD // nw
    mesh = plsc.VectorSubcoreMesh(core_axis_name="c", subcore_axis_name="s")

    @functools.partial(
        pl.kernel, mesh=mesh,
        out_type=jax.ShapeDtypeStruct((_PAD, _TD), jnp.float32),
        scratch_types=[
            pltpu.VMEM((b_per_w,), jnp.int32),
            pltpu.VMEM((b_per_w, _TD), jnp.float32),
            pltpu.SemaphoreType.DMA,
        ],
    )
    def gather_k(table_hbm, idx_hbm, out_hbm, idx_v, rows_v, sem):
        wid = lax.axis_index("s") * info.num_cores + lax.axis_index("c")
        base = wid * b_per_w
        pltpu.sync_copy(idx_hbm.at[pl.ds(base, b_per_w)], idx_v)
        pltpu.async_copy(table_hbm.at[idx_v], rows_v, sem).wait()
        pltpu.sync_copy(rows_v, out_hbm.at[pl.ds(base, b_per_w)])

    return gather_k(table, idx)


@jax.jit
def kernel(rpn_box, rpn_prob, anchors):
    objness = 1.0 - rpn_prob[:, 0]
    scores, inds = jax.lax.top_k(objness, _TOP_N)

    n = rpn_box.shape[0]
    table = jnp.concatenate(
        [rpn_box, anchors, jnp.zeros((n, _TD - 8), jnp.float32)], axis=1)
    idx = jnp.pad(inds, (0, _PAD - _TOP_N)).astype(jnp.int32)
    rows = _sc_gather_rows(table, idx)

    ins = [rows[:, j].reshape(_ROWS, _COLS) for j in range(8)]

    shp = jax.ShapeDtypeStruct((_ROWS, _COLS), jnp.float32)
    x1, y1, x2, y2, keepf = pl.pallas_call(
        _decode_nms_body,
        out_shape=[shp] * 5,
    )(*ins)

    decoded = jnp.stack(
        [x1.reshape(-1), y1.reshape(-1), x2.reshape(-1), y2.reshape(-1)],
        axis=1)[:_TOP_N]
    keep = keepf.reshape(-1)[:_TOP_N] > 0.5
    masked = jnp.where(keep, scores, -1e9)
    final_scores, fi = jax.lax.top_k(masked, _TOP_N_POST)
    final_boxes = jnp.take(decoded, fi, axis=0)
    return jnp.concatenate([final_boxes, final_scores[:, None]], axis=1)
